# baseline (speedup 1.0000x reference)
import jax
import jax.numpy as jnp
import numpy as np
from jax.experimental import pallas as pl
from jax.experimental.pallas import tpu as pltpu

N = 4096
D_IN = 256
H = 128
C = 32
CP = C + 1
K = 4
LAM_HAT = 0.9
A_SCAD = 3.7
EPS = 1e-8

BT = 512
NT = N // BT
NPAIRS = NT * (NT + 1) // 2
BP = 256


def _prep_kernel(A_ref, F_ref, W1_ref, b1_ref, W2_ref, b2_ref,
                 F0_ref, Dd_ref, dinv_ref):
    a = A_ref[...]
    dd = jnp.sum(a, axis=1, keepdims=True) + 1.0
    Dd_ref[...] = dd
    dinv_ref[...] = jax.lax.rsqrt(dd)
    h = jnp.maximum(
        jnp.dot(F_ref[...], W1_ref[...], preferred_element_type=jnp.float32)
        + b1_ref[...], 0.0)
    F0_ref[...] = (jnp.dot(h, W2_ref[...], preferred_element_type=jnp.float32)
                   + b2_ref[...])


def _iter_kernel(ti_ref, tj_ref, lam_ref, A1_ref, At1_ref, Fc_ref, dinv_ref,
                 Dd_ref, F0_ref, out_ref, S_acc, P_acc, P2T_acc):
    p = pl.program_id(0)
    ti = ti_ref[p]
    tj = tj_ref[p]
    lam_k = lam_ref[0]
    lam = 1.0 / LAM_HAT - 1.0
    alam = A_SCAD * lam_k
    inv_c = 1.0 / (2.0 * (A_SCAD - 1.0) * lam_k)

    @pl.when(p == 0)
    def _():
        S_acc[...] = jnp.zeros_like(S_acc)
        P_acc[...] = jnp.zeros_like(P_acc)
        P2T_acc[...] = jnp.zeros_like(P2T_acc)

    @pl.when(p < NPAIRS)
    def _():
        xni = Fc_ref[pl.ds(ti * BT, BT), :] * dinv_ref[pl.ds(ti * BT, BT), :]
        xnj = Fc_ref[pl.ds(tj * BT, BT), :] * dinv_ref[pl.ds(tj * BT, BT), :]
        sqi = jnp.sum(xni * xni, axis=1, keepdims=True)
        sqj = jnp.sum(xnj * xnj, axis=1, keepdims=True).T

        g = jax.lax.dot_general(xni, xnj, (((1,), (1,)), ((), ())),
                                preferred_element_type=jnp.float32)
        z = jnp.maximum(sqi + sqj - 2.0 * g, 0.0)
        r = jax.lax.rsqrt(jnp.maximum(z, EPS * EPS))
        y = z * r
        t = jnp.maximum(jnp.minimum(alam * inv_c - y * inv_c, 0.5), 0.0)
        w = t * r

        @pl.when(ti == tj)
        def _():
            row = jax.lax.broadcasted_iota(jnp.int32, (BT, BT), 0)
            col = jax.lax.broadcasted_iota(jnp.int32, (BT, BT), 1)
            wd = jnp.where(row == col, 0.0, w)
            wa1 = wd * A1_ref[...]
            S_acc[pl.ds(ti * BT, BT), :] += jnp.sum(wa1, axis=1,
                                                    keepdims=True)
            P_acc[pl.ds(ti * BT, BT), :] += jax.lax.dot_general(
                wa1, xnj, (((1,), (0,)), ((), ())),
                preferred_element_type=jnp.float32)

        @pl.when(ti != tj)
        def _():
            wa1 = w * A1_ref[...]
            S_acc[pl.ds(ti * BT, BT), :] += jnp.sum(wa1, axis=1,
                                                    keepdims=True)
            P_acc[pl.ds(ti * BT, BT), :] += jax.lax.dot_general(
                wa1, xnj, (((1,), (0,)), ((), ())),
                preferred_element_type=jnp.float32)
            u = w * At1_ref[...]
            biT = jnp.concatenate([xni.T, jnp.ones((1, BT), jnp.float32)],
                                  axis=0)
            P2T_acc[:, pl.ds(tj * BT, BT)] += jax.lax.dot_general(
                biT, u, (((1,), (0,)), ((), ())),
                preferred_element_type=jnp.float32)

    @pl.when(p == NPAIRS)
    def _():
        p2 = P2T_acc[...].T
        s_tot = S_acc[...] + p2[:, C:C + 1]
        q = s_tot / Dd_ref[...] + lam
        fc = (dinv_ref[...] * (P_acc[...] + p2[:, :C])
              + lam * F0_ref[...]) / q
        out_ref[...] = fc


def _prep_call(A, F, W1, b1, W2, b2):
    return pl.pallas_call(
        _prep_kernel,
        grid=(N // BP,),
        in_specs=[
            pl.BlockSpec((BP, N), lambda i: (i, 0)),
            pl.BlockSpec((BP, D_IN), lambda i: (i, 0)),
            pl.BlockSpec((D_IN, H), lambda i: (0, 0)),
            pl.BlockSpec((1, H), lambda i: (0, 0)),
            pl.BlockSpec((H, C), lambda i: (0, 0)),
            pl.BlockSpec((1, C), lambda i: (0, 0)),
        ],
        out_specs=[
            pl.BlockSpec((BP, C), lambda i: (i, 0)),
            pl.BlockSpec((BP, 1), lambda i: (i, 0)),
            pl.BlockSpec((BP, 1), lambda i: (i, 0)),
        ],
        out_shape=[
            jax.ShapeDtypeStruct((N, C), jnp.float32),
            jax.ShapeDtypeStruct((N, 1), jnp.float32),
            jax.ShapeDtypeStruct((N, 1), jnp.float32),
        ],
        compiler_params=pltpu.CompilerParams(
            dimension_semantics=("arbitrary",)),
    )(A, F, W1, b1, W2, b2)


_TI_LIST = []
_TJ_LIST = []
for _a in range(NT):
    for _b in range(_a, NT):
        _TI_LIST.append(_a)
        _TJ_LIST.append(_b)
_TI_LIST.append(0)
_TJ_LIST.append(0)
_TI_ARR = np.asarray(_TI_LIST, np.int32)
_TJ_ARR = np.asarray(_TJ_LIST, np.int32)


def _iter_call(lam_k, A, At, Fc, dinv, Dd, F0):
    grid_spec = pltpu.PrefetchScalarGridSpec(
        num_scalar_prefetch=3,
        grid=(NPAIRS + 1,),
        in_specs=[
            pl.BlockSpec((BT, BT), lambda p, ti, tj, lam: (ti[p], tj[p])),
            pl.BlockSpec((BT, BT), lambda p, ti, tj, lam: (ti[p], tj[p])),
            pl.BlockSpec((N, C), lambda p, ti, tj, lam: (0, 0)),
            pl.BlockSpec((N, 1), lambda p, ti, tj, lam: (0, 0)),
            pl.BlockSpec((N, 1), lambda p, ti, tj, lam: (0, 0)),
            pl.BlockSpec((N, C), lambda p, ti, tj, lam: (0, 0)),
        ],
        out_specs=pl.BlockSpec((N, C), lambda p, ti, tj, lam: (0, 0)),
        scratch_shapes=[
            pltpu.VMEM((N, 1), jnp.float32),
            pltpu.VMEM((N, C), jnp.float32),
            pltpu.VMEM((CP, N), jnp.float32),
        ],
    )
    return pl.pallas_call(
        _iter_kernel,
        grid_spec=grid_spec,
        out_shape=jax.ShapeDtypeStruct((N, C), jnp.float32),
        compiler_params=pltpu.CompilerParams(
            dimension_semantics=("arbitrary",)),
    )(jnp.asarray(_TI_ARR), jnp.asarray(_TJ_ARR), lam_k,
      A, At, Fc, dinv, Dd, F0)


def kernel(A, F, W1, b1, W2, b2, log_lams):
    F0, Dd, dinv = _prep_call(A, F, W1, b1.reshape(1, H), W2, b2.reshape(1, C))
    At = A.T
    lams = jnp.exp(log_lams)
    Fc = F0
    for k in range(K):
        Fc = _iter_call(lams[k].reshape(1), A, At, Fc, dinv, Dd, F0)
    return Fc


# speedup vs baseline: 1.2308x; 1.2308x over previous
import jax
import jax.numpy as jnp
import numpy as np
from jax.experimental import pallas as pl
from jax.experimental.pallas import tpu as pltpu

N = 4096
D_IN = 256
H = 128
C = 32
K = 4
LAM_HAT = 0.9
A_SCAD = 3.7
EPS = 1e-8

BT = 512
NT = N // BT
NPAIRS = NT * (NT + 1) // 2
BP = 256


def _prep_kernel(A_ref, F_ref, W1_ref, b1_ref, W2_ref, b2_ref,
                 F0_ref, Dd_ref, dinv_ref, Ab_ref):
    a = A_ref[...]
    dd = jnp.sum(a, axis=1, keepdims=True) + 1.0
    Dd_ref[...] = dd
    dinv_ref[...] = jax.lax.rsqrt(dd)
    Ab_ref[...] = a.astype(jnp.bfloat16)
    h = jnp.maximum(
        jnp.dot(F_ref[...], W1_ref[...], preferred_element_type=jnp.float32)
        + b1_ref[...], 0.0)
    F0_ref[...] = (jnp.dot(h, W2_ref[...], preferred_element_type=jnp.float32)
                   + b2_ref[...])


def _iter_kernel(ti_ref, tj_ref, lam_ref, A1_ref, A2_ref, Fc_ref, dinv_ref,
                 Dd_ref, F0_ref, out_ref, S_acc, P_acc):
    p = pl.program_id(0)
    ti = ti_ref[p]
    tj = tj_ref[p]
    lam_k = lam_ref[0]
    lam = 1.0 / LAM_HAT - 1.0
    alam = A_SCAD * lam_k
    inv_c = 1.0 / (2.0 * (A_SCAD - 1.0) * lam_k)

    @pl.when(p == 0)
    def _():
        S_acc[...] = jnp.zeros_like(S_acc)
        P_acc[...] = jnp.zeros_like(P_acc)

    @pl.when(p < NPAIRS)
    def _():
        xni = Fc_ref[pl.ds(ti * BT, BT), :] * dinv_ref[pl.ds(ti * BT, BT), :]
        xnj = Fc_ref[pl.ds(tj * BT, BT), :] * dinv_ref[pl.ds(tj * BT, BT), :]
        sqi = jnp.sum(xni * xni, axis=1, keepdims=True)
        sqj = jnp.sum(xnj * xnj, axis=1, keepdims=True).T
        xnib = xni.astype(jnp.bfloat16)
        xnjb = xnj.astype(jnp.bfloat16)

        g = jax.lax.dot_general(xni, xnj, (((1,), (1,)), ((), ())),
                                preferred_element_type=jnp.float32)
        zpre = sqi + sqj - 2.0 * g
        r = jax.lax.rsqrt(jnp.maximum(zpre, EPS * EPS))
        y = zpre * r
        t = jnp.maximum(jnp.minimum(alam * inv_c - y * inv_c, 0.5), 0.0)
        w = (t * r).astype(jnp.bfloat16)

        @pl.when(ti == tj)
        def _():
            row = jax.lax.broadcasted_iota(jnp.int32, (BT, BT), 0)
            col = jax.lax.broadcasted_iota(jnp.int32, (BT, BT), 1)
            wd = jnp.where(row == col, jnp.bfloat16(0.0), w)
            wa1 = wd * A1_ref[...]
            S_acc[pl.ds(ti * BT, BT), :] += jnp.sum(
                wa1, axis=1, keepdims=True, dtype=jnp.float32)
            P_acc[pl.ds(ti * BT, BT), :] += jax.lax.dot_general(
                wa1, xnjb, (((1,), (0,)), ((), ())),
                preferred_element_type=jnp.float32)

        @pl.when(ti != tj)
        def _():
            wa1 = w * A1_ref[...]
            S_acc[pl.ds(ti * BT, BT), :] += jnp.sum(
                wa1, axis=1, keepdims=True, dtype=jnp.float32)
            P_acc[pl.ds(ti * BT, BT), :] += jax.lax.dot_general(
                wa1, xnjb, (((1,), (0,)), ((), ())),
                preferred_element_type=jnp.float32)
            wa2 = w.T * A2_ref[...]
            S_acc[pl.ds(tj * BT, BT), :] += jnp.sum(
                wa2, axis=1, keepdims=True, dtype=jnp.float32)
            P_acc[pl.ds(tj * BT, BT), :] += jax.lax.dot_general(
                wa2, xnib, (((1,), (0,)), ((), ())),
                preferred_element_type=jnp.float32)

    @pl.when(p == NPAIRS)
    def _():
        q = S_acc[...] / Dd_ref[...] + lam
        out_ref[...] = (dinv_ref[...] * P_acc[...] + lam * F0_ref[...]) / q


def _prep_call(A, F, W1, b1, W2, b2):
    return pl.pallas_call(
        _prep_kernel,
        grid=(N // BP,),
        in_specs=[
            pl.BlockSpec((BP, N), lambda i: (i, 0)),
            pl.BlockSpec((BP, D_IN), lambda i: (i, 0)),
            pl.BlockSpec((D_IN, H), lambda i: (0, 0)),
            pl.BlockSpec((1, H), lambda i: (0, 0)),
            pl.BlockSpec((H, C), lambda i: (0, 0)),
            pl.BlockSpec((1, C), lambda i: (0, 0)),
        ],
        out_specs=[
            pl.BlockSpec((BP, C), lambda i: (i, 0)),
            pl.BlockSpec((BP, 1), lambda i: (i, 0)),
            pl.BlockSpec((BP, 1), lambda i: (i, 0)),
            pl.BlockSpec((BP, N), lambda i: (i, 0)),
        ],
        out_shape=[
            jax.ShapeDtypeStruct((N, C), jnp.float32),
            jax.ShapeDtypeStruct((N, 1), jnp.float32),
            jax.ShapeDtypeStruct((N, 1), jnp.float32),
            jax.ShapeDtypeStruct((N, N), jnp.bfloat16),
        ],
        compiler_params=pltpu.CompilerParams(
            dimension_semantics=("arbitrary",)),
    )(A, F, W1, b1, W2, b2)


_TI_LIST = []
_TJ_LIST = []
for _a in range(NT):
    for _b in range(_a, NT):
        _TI_LIST.append(_a)
        _TJ_LIST.append(_b)
_TI_LIST.append(0)
_TJ_LIST.append(0)
_TI_ARR = np.asarray(_TI_LIST, np.int32)
_TJ_ARR = np.asarray(_TJ_LIST, np.int32)


def _iter_call(lam_k, Ab, Fc, dinv, Dd, F0):
    grid_spec = pltpu.PrefetchScalarGridSpec(
        num_scalar_prefetch=3,
        grid=(NPAIRS + 1,),
        in_specs=[
            pl.BlockSpec((BT, BT), lambda p, ti, tj, lam: (ti[p], tj[p])),
            pl.BlockSpec((BT, BT), lambda p, ti, tj, lam: (tj[p], ti[p])),
            pl.BlockSpec((N, C), lambda p, ti, tj, lam: (0, 0)),
            pl.BlockSpec((N, 1), lambda p, ti, tj, lam: (0, 0)),
            pl.BlockSpec((N, 1), lambda p, ti, tj, lam: (0, 0)),
            pl.BlockSpec((N, C), lambda p, ti, tj, lam: (0, 0)),
        ],
        out_specs=pl.BlockSpec((N, C), lambda p, ti, tj, lam: (0, 0)),
        scratch_shapes=[
            pltpu.VMEM((N, 1), jnp.float32),
            pltpu.VMEM((N, C), jnp.float32),
        ],
    )
    return pl.pallas_call(
        _iter_kernel,
        grid_spec=grid_spec,
        out_shape=jax.ShapeDtypeStruct((N, C), jnp.float32),
        compiler_params=pltpu.CompilerParams(
            dimension_semantics=("arbitrary",)),
    )(jnp.asarray(_TI_ARR), jnp.asarray(_TJ_ARR), lam_k,
      Ab, Ab, Fc, dinv, Dd, F0)


def kernel(A, F, W1, b1, W2, b2, log_lams):
    F0, Dd, dinv, Ab = _prep_call(
        A, F, W1, b1.reshape(1, H), W2, b2.reshape(1, C))
    lams = jnp.exp(log_lams)
    Fc = F0
    for k in range(K):
        Fc = _iter_call(lams[k].reshape(1), Ab, Fc, dinv, Dd, F0)
    return Fc


# speedup vs baseline: 1.2645x; 1.0274x over previous
import jax
import jax.numpy as jnp
import numpy as np
from jax.experimental import pallas as pl
from jax.experimental.pallas import tpu as pltpu

N = 4096
D_IN = 256
H = 128
C = 32
K = 4
LAM_HAT = 0.9
A_SCAD = 3.7
EPS = 1e-8

BT = 512
NT = N // BT
NPAIRS = NT * (NT + 1) // 2
BP = 256


def _prep_kernel(A_ref, F_ref, W1_ref, b1_ref, W2_ref, b2_ref,
                 F0_ref, Dd_ref, dinv_ref, Ab_ref):
    a = A_ref[...]
    dd = jnp.sum(a, axis=1, keepdims=True) + 1.0
    Dd_ref[...] = dd
    dinv_ref[...] = jax.lax.rsqrt(dd)
    Ab_ref[...] = a.astype(jnp.bfloat16)
    h = jnp.maximum(
        jnp.dot(F_ref[...], W1_ref[...], preferred_element_type=jnp.float32)
        + b1_ref[...], 0.0)
    F0_ref[...] = (jnp.dot(h, W2_ref[...], preferred_element_type=jnp.float32)
                   + b2_ref[...])


def _iter_kernel(ti_ref, tj_ref, lam_ref, A1_ref, A2_ref, Fc_ref, dinv_ref,
                 Dd_ref, F0_ref, out_ref, P_acc):
    p = pl.program_id(0)
    ti = ti_ref[p]
    tj = tj_ref[p]
    lam_k = lam_ref[0]
    lam = 1.0 / LAM_HAT - 1.0
    alam = A_SCAD * lam_k
    inv_c = 1.0 / (2.0 * (A_SCAD - 1.0) * lam_k)

    @pl.when(p == 0)
    def _():
        P_acc[...] = jnp.zeros_like(P_acc)

    @pl.when(p < NPAIRS)
    def _():
        xni = Fc_ref[pl.ds(ti * BT, BT), :] * dinv_ref[pl.ds(ti * BT, BT), :]
        xnj = Fc_ref[pl.ds(tj * BT, BT), :] * dinv_ref[pl.ds(tj * BT, BT), :]
        sqi = jnp.sum(xni * xni, axis=1, keepdims=True)
        sqj = jnp.sum(xnj * xnj, axis=1, keepdims=True).T
        onesb = jnp.ones((BT, 1), jnp.bfloat16)
        xnib = jnp.concatenate([xni.astype(jnp.bfloat16), onesb], axis=1)
        xnjb = jnp.concatenate([xnj.astype(jnp.bfloat16), onesb], axis=1)

        g = jax.lax.dot_general(xni, xnj, (((1,), (1,)), ((), ())),
                                preferred_element_type=jnp.float32)
        zpre = sqi + sqj - 2.0 * g
        r = jax.lax.rsqrt(jnp.maximum(zpre, EPS * EPS))
        y = zpre * r
        t = jnp.maximum(jnp.minimum(alam * inv_c - y * inv_c, 0.5), 0.0)
        w = (t * r).astype(jnp.bfloat16)

        @pl.when(ti == tj)
        def _():
            row = jax.lax.broadcasted_iota(jnp.int32, (BT, BT), 0)
            col = jax.lax.broadcasted_iota(jnp.int32, (BT, BT), 1)
            wd = jnp.where(row == col, jnp.bfloat16(0.0), w)
            wa1 = wd * A1_ref[...]
            P_acc[pl.ds(ti * BT, BT), :] += jax.lax.dot_general(
                wa1, xnjb, (((1,), (0,)), ((), ())),
                preferred_element_type=jnp.float32)

        @pl.when(ti != tj)
        def _():
            wa1 = w * A1_ref[...]
            P_acc[pl.ds(ti * BT, BT), :] += jax.lax.dot_general(
                wa1, xnjb, (((1,), (0,)), ((), ())),
                preferred_element_type=jnp.float32)
            wa2 = w.T * A2_ref[...]
            P_acc[pl.ds(tj * BT, BT), :] += jax.lax.dot_general(
                wa2, xnib, (((1,), (0,)), ((), ())),
                preferred_element_type=jnp.float32)

    @pl.when(p == NPAIRS)
    def _():
        q = P_acc[:, C:C + 1] / Dd_ref[...] + lam
        out_ref[...] = (dinv_ref[...] * P_acc[:, :C] + lam * F0_ref[...]) / q


def _prep_call(A, F, W1, b1, W2, b2):
    return pl.pallas_call(
        _prep_kernel,
        grid=(N // BP,),
        in_specs=[
            pl.BlockSpec((BP, N), lambda i: (i, 0)),
            pl.BlockSpec((BP, D_IN), lambda i: (i, 0)),
            pl.BlockSpec((D_IN, H), lambda i: (0, 0)),
            pl.BlockSpec((1, H), lambda i: (0, 0)),
            pl.BlockSpec((H, C), lambda i: (0, 0)),
            pl.BlockSpec((1, C), lambda i: (0, 0)),
        ],
        out_specs=[
            pl.BlockSpec((BP, C), lambda i: (i, 0)),
            pl.BlockSpec((BP, 1), lambda i: (i, 0)),
            pl.BlockSpec((BP, 1), lambda i: (i, 0)),
            pl.BlockSpec((BP, N), lambda i: (i, 0)),
        ],
        out_shape=[
            jax.ShapeDtypeStruct((N, C), jnp.float32),
            jax.ShapeDtypeStruct((N, 1), jnp.float32),
            jax.ShapeDtypeStruct((N, 1), jnp.float32),
            jax.ShapeDtypeStruct((N, N), jnp.bfloat16),
        ],
        compiler_params=pltpu.CompilerParams(
            dimension_semantics=("arbitrary",)),
    )(A, F, W1, b1, W2, b2)


_TI_LIST = []
_TJ_LIST = []
for _a in range(NT):
    for _b in range(_a, NT):
        _TI_LIST.append(_a)
        _TJ_LIST.append(_b)
_TI_LIST.append(0)
_TJ_LIST.append(0)
_TI_ARR = np.asarray(_TI_LIST, np.int32)
_TJ_ARR = np.asarray(_TJ_LIST, np.int32)


def _iter_call(lam_k, Ab, Fc, dinv, Dd, F0):
    grid_spec = pltpu.PrefetchScalarGridSpec(
        num_scalar_prefetch=3,
        grid=(NPAIRS + 1,),
        in_specs=[
            pl.BlockSpec((BT, BT), lambda p, ti, tj, lam: (ti[p], tj[p])),
            pl.BlockSpec((BT, BT), lambda p, ti, tj, lam: (tj[p], ti[p])),
            pl.BlockSpec((N, C), lambda p, ti, tj, lam: (0, 0)),
            pl.BlockSpec((N, 1), lambda p, ti, tj, lam: (0, 0)),
            pl.BlockSpec((N, 1), lambda p, ti, tj, lam: (0, 0)),
            pl.BlockSpec((N, C), lambda p, ti, tj, lam: (0, 0)),
        ],
        out_specs=pl.BlockSpec((N, C), lambda p, ti, tj, lam: (0, 0)),
        scratch_shapes=[
            pltpu.VMEM((N, C + 1), jnp.float32),
        ],
    )
    return pl.pallas_call(
        _iter_kernel,
        grid_spec=grid_spec,
        out_shape=jax.ShapeDtypeStruct((N, C), jnp.float32),
        compiler_params=pltpu.CompilerParams(
            dimension_semantics=("arbitrary",)),
    )(jnp.asarray(_TI_ARR), jnp.asarray(_TJ_ARR), lam_k,
      Ab, Ab, Fc, dinv, Dd, F0)


def kernel(A, F, W1, b1, W2, b2, log_lams):
    F0, Dd, dinv, Ab = _prep_call(
        A, F, W1, b1.reshape(1, H), W2, b2.reshape(1, C))
    lams = jnp.exp(log_lams)
    Fc = F0
    for k in range(K):
        Fc = _iter_call(lams[k].reshape(1), Ab, Fc, dinv, Dd, F0)
    return Fc


# speedup vs baseline: 1.3028x; 1.0303x over previous
import jax
import jax.numpy as jnp
import numpy as np
from jax.experimental import pallas as pl
from jax.experimental.pallas import tpu as pltpu

N = 4096
D_IN = 256
H = 128
C = 32
K = 4
LAM_HAT = 0.9
A_SCAD = 3.7
EPS = 1e-8

BT = 512
NT = N // BT
NPAIRS = NT * (NT + 1) // 2
BP = 256


def _prep_kernel(A_ref, F_ref, W1_ref, b1_ref, W2_ref, b2_ref,
                 F0_ref, Dd_ref, dinv_ref, Ab_ref):
    a = A_ref[...]
    dd = jnp.sum(a, axis=1, keepdims=True) + 1.0
    Dd_ref[...] = dd
    dinv_ref[...] = jax.lax.rsqrt(dd)
    Ab_ref[...] = a.astype(jnp.bfloat16)
    h = jnp.maximum(
        jnp.dot(F_ref[...], W1_ref[...], preferred_element_type=jnp.float32)
        + b1_ref[...], 0.0)
    F0_ref[...] = (jnp.dot(h, W2_ref[...], preferred_element_type=jnp.float32)
                   + b2_ref[...])


def _iter_kernel(ti_ref, tj_ref, lam_ref, A1_ref, A2_ref, Fc_ref, dinv_ref,
                 Dd_ref, F0_ref, out_ref, P_acc):
    p = pl.program_id(0)
    ti = ti_ref[p]
    tj = tj_ref[p]
    lam_k = lam_ref[0]
    lam = 1.0 / LAM_HAT - 1.0
    alam = A_SCAD * lam_k
    inv_c = 1.0 / (2.0 * (A_SCAD - 1.0) * lam_k)

    @pl.when(p == 0)
    def _():
        P_acc[...] = jnp.zeros_like(P_acc)

    @pl.when(p < NPAIRS)
    def _():
        xni = Fc_ref[pl.ds(ti * BT, BT), :] * dinv_ref[pl.ds(ti * BT, BT), :]
        xnj = Fc_ref[pl.ds(tj * BT, BT), :] * dinv_ref[pl.ds(tj * BT, BT), :]
        sqi = jnp.sum(xni * xni, axis=1, keepdims=True)
        sqj = jnp.sum(xnj * xnj, axis=1, keepdims=True)
        onesf = jnp.ones((BT, 1), jnp.float32)
        onesb = jnp.ones((BT, 1), jnp.bfloat16)
        xnib = jnp.concatenate([xni.astype(jnp.bfloat16), onesb], axis=1)
        xnjb = jnp.concatenate([xnj.astype(jnp.bfloat16), onesb], axis=1)

        ai = jnp.concatenate([xni * -2.0, onesf], axis=1)
        bj = jnp.concatenate([xnj, sqj], axis=1)
        gz = jax.lax.dot_general(ai, bj, (((1,), (1,)), ((), ())),
                                 preferred_element_type=jnp.float32)
        zpre = sqi + gz
        r = jax.lax.rsqrt(jnp.maximum(zpre, EPS * EPS))
        y = zpre * r
        t = jnp.maximum(jnp.minimum(alam * inv_c - y * inv_c, 0.5), 0.0)
        w = (t * r).astype(jnp.bfloat16)

        @pl.when(ti == tj)
        def _():
            row = jax.lax.broadcasted_iota(jnp.int32, (BT, BT), 0)
            col = jax.lax.broadcasted_iota(jnp.int32, (BT, BT), 1)
            wd = jnp.where(row == col, jnp.bfloat16(0.0), w)
            wa1 = wd * A1_ref[...]
            P_acc[pl.ds(ti * BT, BT), :] += jax.lax.dot_general(
                wa1, xnjb, (((1,), (0,)), ((), ())),
                preferred_element_type=jnp.float32)

        @pl.when(ti != tj)
        def _():
            wa1 = w * A1_ref[...]
            P_acc[pl.ds(ti * BT, BT), :] += jax.lax.dot_general(
                wa1, xnjb, (((1,), (0,)), ((), ())),
                preferred_element_type=jnp.float32)
            wa2 = w.T * A2_ref[...]
            P_acc[pl.ds(tj * BT, BT), :] += jax.lax.dot_general(
                wa2, xnib, (((1,), (0,)), ((), ())),
                preferred_element_type=jnp.float32)

    @pl.when(p == NPAIRS)
    def _():
        q = P_acc[:, C:C + 1] / Dd_ref[...] + lam
        out_ref[...] = (dinv_ref[...] * P_acc[:, :C] + lam * F0_ref[...]) / q


def _prep_call(A, F, W1, b1, W2, b2):
    return pl.pallas_call(
        _prep_kernel,
        grid=(N // BP,),
        in_specs=[
            pl.BlockSpec((BP, N), lambda i: (i, 0)),
            pl.BlockSpec((BP, D_IN), lambda i: (i, 0)),
            pl.BlockSpec((D_IN, H), lambda i: (0, 0)),
            pl.BlockSpec((1, H), lambda i: (0, 0)),
            pl.BlockSpec((H, C), lambda i: (0, 0)),
            pl.BlockSpec((1, C), lambda i: (0, 0)),
        ],
        out_specs=[
            pl.BlockSpec((BP, C), lambda i: (i, 0)),
            pl.BlockSpec((BP, 1), lambda i: (i, 0)),
            pl.BlockSpec((BP, 1), lambda i: (i, 0)),
            pl.BlockSpec((BP, N), lambda i: (i, 0)),
        ],
        out_shape=[
            jax.ShapeDtypeStruct((N, C), jnp.float32),
            jax.ShapeDtypeStruct((N, 1), jnp.float32),
            jax.ShapeDtypeStruct((N, 1), jnp.float32),
            jax.ShapeDtypeStruct((N, N), jnp.bfloat16),
        ],
        compiler_params=pltpu.CompilerParams(
            dimension_semantics=("arbitrary",)),
    )(A, F, W1, b1, W2, b2)


_TI_LIST = []
_TJ_LIST = []
for _a in range(NT):
    for _b in range(_a, NT):
        _TI_LIST.append(_a)
        _TJ_LIST.append(_b)
_TI_LIST.append(0)
_TJ_LIST.append(0)
_TI_ARR = np.asarray(_TI_LIST, np.int32)
_TJ_ARR = np.asarray(_TJ_LIST, np.int32)


def _iter_call(lam_k, Ab, Fc, dinv, Dd, F0):
    grid_spec = pltpu.PrefetchScalarGridSpec(
        num_scalar_prefetch=3,
        grid=(NPAIRS + 1,),
        in_specs=[
            pl.BlockSpec((BT, BT), lambda p, ti, tj, lam: (ti[p], tj[p])),
            pl.BlockSpec((BT, BT), lambda p, ti, tj, lam: (tj[p], ti[p])),
            pl.BlockSpec((N, C), lambda p, ti, tj, lam: (0, 0)),
            pl.BlockSpec((N, 1), lambda p, ti, tj, lam: (0, 0)),
            pl.BlockSpec((N, 1), lambda p, ti, tj, lam: (0, 0)),
            pl.BlockSpec((N, C), lambda p, ti, tj, lam: (0, 0)),
        ],
        out_specs=pl.BlockSpec((N, C), lambda p, ti, tj, lam: (0, 0)),
        scratch_shapes=[
            pltpu.VMEM((N, C + 1), jnp.float32),
        ],
    )
    return pl.pallas_call(
        _iter_kernel,
        grid_spec=grid_spec,
        out_shape=jax.ShapeDtypeStruct((N, C), jnp.float32),
        compiler_params=pltpu.CompilerParams(
            dimension_semantics=("arbitrary",)),
    )(jnp.asarray(_TI_ARR), jnp.asarray(_TJ_ARR), lam_k,
      Ab, Ab, Fc, dinv, Dd, F0)


def kernel(A, F, W1, b1, W2, b2, log_lams):
    F0, Dd, dinv, Ab = _prep_call(
        A, F, W1, b1.reshape(1, H), W2, b2.reshape(1, C))
    lams = jnp.exp(log_lams)
    Fc = F0
    for k in range(K):
        Fc = _iter_call(lams[k].reshape(1), Ab, Fc, dinv, Dd, F0)
    return Fc


# speedup vs baseline: 1.3692x; 1.0510x over previous
import jax
import jax.numpy as jnp
import numpy as np
from jax.experimental import pallas as pl
from jax.experimental.pallas import tpu as pltpu

N = 4096
D_IN = 256
H = 128
C = 32
K = 4
LAM_HAT = 0.9
A_SCAD = 3.7
EPS = 1e-8

BT = 512
NT = N // BT
NPAIRS = NT * (NT + 1) // 2
BP = 256


def _prep_kernel(A_ref, F_ref, W1_ref, b1_ref, W2_ref, b2_ref,
                 F0_ref, Dd_ref, dinv_ref, Ab_ref):
    a = A_ref[...]
    dd = jnp.sum(a, axis=1, keepdims=True) + 1.0
    Dd_ref[...] = dd
    dinv_ref[...] = jax.lax.rsqrt(dd)
    Ab_ref[...] = a.astype(jnp.bfloat16)
    h = jnp.maximum(
        jnp.dot(F_ref[...], W1_ref[...], preferred_element_type=jnp.float32)
        + b1_ref[...], 0.0)
    F0_ref[...] = (jnp.dot(h, W2_ref[...], preferred_element_type=jnp.float32)
                   + b2_ref[...])


def _iter_kernel(ti_ref, tj_ref, lam_ref, A1_ref, A2_ref, Fc_ref, dinv_ref,
                 Dd_ref, F0_ref, out_ref, P_acc, AI_s, BJ_s, XB_s, SQ_s):
    p = pl.program_id(0)
    ti = ti_ref[p]
    tj = tj_ref[p]
    lam_k = lam_ref[0]
    lam = 1.0 / LAM_HAT - 1.0
    alam = A_SCAD * lam_k
    inv_c = 1.0 / (2.0 * (A_SCAD - 1.0) * lam_k)

    @pl.when(p == 0)
    def _():
        xn = Fc_ref[...] * dinv_ref[...]
        sq = jnp.sum(xn * xn, axis=1, keepdims=True)
        onesf = jnp.ones((N, 1), jnp.float32)
        onesb = jnp.ones((N, 1), jnp.bfloat16)
        AI_s[...] = jnp.concatenate([xn * -2.0, onesf], axis=1)
        BJ_s[...] = jnp.concatenate([xn, sq], axis=1)
        XB_s[...] = jnp.concatenate([xn.astype(jnp.bfloat16), onesb], axis=1)
        SQ_s[...] = sq
        P_acc[...] = jnp.zeros_like(P_acc)

    @pl.when(p < NPAIRS)
    def _():
        sqi = SQ_s[pl.ds(ti * BT, BT), :]
        xnib = XB_s[pl.ds(ti * BT, BT), :]
        xnjb = XB_s[pl.ds(tj * BT, BT), :]

        ai = AI_s[pl.ds(ti * BT, BT), :]
        bj = BJ_s[pl.ds(tj * BT, BT), :]
        gz = jax.lax.dot_general(ai, bj, (((1,), (1,)), ((), ())),
                                 preferred_element_type=jnp.float32)
        zpre = sqi + gz
        r = jax.lax.rsqrt(jnp.maximum(zpre, EPS * EPS))
        y = zpre * r
        t = jnp.maximum(jnp.minimum(alam * inv_c - y * inv_c, 0.5), 0.0)
        w = (t * r).astype(jnp.bfloat16)

        @pl.when(ti == tj)
        def _():
            row = jax.lax.broadcasted_iota(jnp.int32, (BT, BT), 0)
            col = jax.lax.broadcasted_iota(jnp.int32, (BT, BT), 1)
            wd = jnp.where(row == col, jnp.bfloat16(0.0), w)
            wa1 = wd * A1_ref[...]
            P_acc[pl.ds(ti * BT, BT), :] += jax.lax.dot_general(
                wa1, xnjb, (((1,), (0,)), ((), ())),
                preferred_element_type=jnp.float32)

        @pl.when(ti != tj)
        def _():
            wa1 = w * A1_ref[...]
            P_acc[pl.ds(ti * BT, BT), :] += jax.lax.dot_general(
                wa1, xnjb, (((1,), (0,)), ((), ())),
                preferred_element_type=jnp.float32)
            wa2 = w.T * A2_ref[...]
            P_acc[pl.ds(tj * BT, BT), :] += jax.lax.dot_general(
                wa2, xnib, (((1,), (0,)), ((), ())),
                preferred_element_type=jnp.float32)

    @pl.when(p == NPAIRS)
    def _():
        q = P_acc[:, C:C + 1] / Dd_ref[...] + lam
        out_ref[...] = (dinv_ref[...] * P_acc[:, :C] + lam * F0_ref[...]) / q


def _prep_call(A, F, W1, b1, W2, b2):
    return pl.pallas_call(
        _prep_kernel,
        grid=(N // BP,),
        in_specs=[
            pl.BlockSpec((BP, N), lambda i: (i, 0)),
            pl.BlockSpec((BP, D_IN), lambda i: (i, 0)),
            pl.BlockSpec((D_IN, H), lambda i: (0, 0)),
            pl.BlockSpec((1, H), lambda i: (0, 0)),
            pl.BlockSpec((H, C), lambda i: (0, 0)),
            pl.BlockSpec((1, C), lambda i: (0, 0)),
        ],
        out_specs=[
            pl.BlockSpec((BP, C), lambda i: (i, 0)),
            pl.BlockSpec((BP, 1), lambda i: (i, 0)),
            pl.BlockSpec((BP, 1), lambda i: (i, 0)),
            pl.BlockSpec((BP, N), lambda i: (i, 0)),
        ],
        out_shape=[
            jax.ShapeDtypeStruct((N, C), jnp.float32),
            jax.ShapeDtypeStruct((N, 1), jnp.float32),
            jax.ShapeDtypeStruct((N, 1), jnp.float32),
            jax.ShapeDtypeStruct((N, N), jnp.bfloat16),
        ],
        compiler_params=pltpu.CompilerParams(
            dimension_semantics=("arbitrary",)),
    )(A, F, W1, b1, W2, b2)


_TI_LIST = []
_TJ_LIST = []
for _a in range(NT):
    for _b in range(_a, NT):
        _TI_LIST.append(_a)
        _TJ_LIST.append(_b)
_TI_LIST.append(0)
_TJ_LIST.append(0)
_TI_ARR = np.asarray(_TI_LIST, np.int32)
_TJ_ARR = np.asarray(_TJ_LIST, np.int32)


def _iter_call(lam_k, Ab, Fc, dinv, Dd, F0):
    grid_spec = pltpu.PrefetchScalarGridSpec(
        num_scalar_prefetch=3,
        grid=(NPAIRS + 1,),
        in_specs=[
            pl.BlockSpec((BT, BT), lambda p, ti, tj, lam: (ti[p], tj[p])),
            pl.BlockSpec((BT, BT), lambda p, ti, tj, lam: (tj[p], ti[p])),
            pl.BlockSpec((N, C), lambda p, ti, tj, lam: (0, 0)),
            pl.BlockSpec((N, 1), lambda p, ti, tj, lam: (0, 0)),
            pl.BlockSpec((N, 1), lambda p, ti, tj, lam: (0, 0)),
            pl.BlockSpec((N, C), lambda p, ti, tj, lam: (0, 0)),
        ],
        out_specs=pl.BlockSpec((N, C), lambda p, ti, tj, lam: (0, 0)),
        scratch_shapes=[
            pltpu.VMEM((N, C + 1), jnp.float32),
            pltpu.VMEM((N, C + 1), jnp.float32),
            pltpu.VMEM((N, C + 1), jnp.float32),
            pltpu.VMEM((N, C + 1), jnp.bfloat16),
            pltpu.VMEM((N, 1), jnp.float32),
        ],
    )
    return pl.pallas_call(
        _iter_kernel,
        grid_spec=grid_spec,
        out_shape=jax.ShapeDtypeStruct((N, C), jnp.float32),
        compiler_params=pltpu.CompilerParams(
            dimension_semantics=("arbitrary",)),
    )(jnp.asarray(_TI_ARR), jnp.asarray(_TJ_ARR), lam_k,
      Ab, Ab, Fc, dinv, Dd, F0)


def kernel(A, F, W1, b1, W2, b2, log_lams):
    F0, Dd, dinv, Ab = _prep_call(
        A, F, W1, b1.reshape(1, H), W2, b2.reshape(1, C))
    lams = jnp.exp(log_lams)
    Fc = F0
    for k in range(K):
        Fc = _iter_call(lams[k].reshape(1), Ab, Fc, dinv, Dd, F0)
    return Fc
